# baseline (speedup 1.0000x reference)
import jax
import jax.numpy as jnp
from jax.experimental import pallas as pl
from jax.experimental.pallas import tpu as pltpu

N = 4096
NV = 16
BI = 512
G = N // BI
BJ = 512
GJ = N // BJ
SD = 40
TSTEPS = G + 1 + G + 1


def _rt_dot(a, b):
    return jax.lax.dot_general(
        a, b, (((1,), (1,)), ((), ())), preferred_element_type=jnp.float32
    )


def _fused_body(adj_ref, oh_ref, colors_ref, counts_ref,
                colsum_ref, t_sig_ref, u_sig_ref, fo_s_ref, iso_s_ref):
    t = pl.program_id(0)

    @pl.when(t < G)
    def _():
        adj = adj_ref[...]
        counts_ref[pl.ds(t * BI, BI), :] = jnp.dot(
            adj, oh_ref[...], preferred_element_type=jnp.float32
        )
        part = jnp.sum(adj, axis=0, keepdims=True)

        @pl.when(t == 0)
        def _():
            colsum_ref[...] = part

        @pl.when(t != 0)
        def _():
            colsum_ref[...] = colsum_ref[...] + part

    @pl.when(t == G)
    def _():
        eye = (
            jax.lax.broadcasted_iota(jnp.int32, (BI, BI), 0)
            == jax.lax.broadcasted_iota(jnp.int32, (BI, BI), 1)
        ).astype(jnp.float32)
        cs_rows = [
            _rt_dot(eye, colsum_ref[:, pl.ds(b * BI, BI)]) for b in range(G)
        ]
        colsum_row = jnp.concatenate(cs_rows, axis=0)

        c = counts_ref[...]
        rowsum = jnp.sum(c, axis=1, keepdims=True)
        iso = (rowsum + colsum_row) == 0
        isof = iso.astype(jnp.float32)
        lane = jax.lax.broadcasted_iota(jnp.int32, (N, NV), 1).astype(
            jnp.float32
        )
        xval = jnp.sum(oh_ref[...] * lane, axis=1, keepdims=True)
        ci = c.astype(jnp.int32)
        hi = (ci >> 6).astype(jnp.float32)
        lo = (ci & 63).astype(jnp.float32)
        nrm = isof + xval * xval + jnp.sum(hi * hi + lo * lo, axis=1,
                                           keepdims=True)
        n2 = jnp.floor(nrm * (1.0 / 4096.0)) * 4096.0
        rem = nrm - n2
        n1 = jnp.floor(rem * (1.0 / 64.0)) * 64.0
        n0 = rem - n1
        ones3 = jnp.ones((N, 3), jnp.float32)
        tmat = jnp.concatenate(
            [2 * isof, 2 * xval, 2 * hi, 2 * lo, -n0, -n1, -n2, ones3], axis=1
        )
        umat = jnp.concatenate(
            [isof, xval, hi, lo, ones3, -n0, -n1, -n2], axis=1
        )
        t_sig_ref[...] = tmat.astype(jnp.bfloat16)
        u_sig_ref[...] = umat.astype(jnp.bfloat16)
        iso_s_ref[...] = iso.astype(jnp.int32)

    @pl.when((t > G) & (t <= 2 * G))
    def _():
        i = t - G - 1
        tb = t_sig_ref[pl.ds(i * BI, BI), :]

        def blockmin(jj):
            g2 = _rt_dot(tb, u_sig_ref[pl.ds(jj * BJ, BJ), :])
            idxj = jax.lax.broadcasted_iota(jnp.int32, (BI, BJ), 1) + jj * BJ
            cand = jnp.where(g2 == 0.0, idxj, jnp.int32(N))
            return jnp.min(cand, axis=1, keepdims=True)

        fo_s_ref[pl.ds(i * BI, BI), :] = blockmin(0)
        for jj in range(1, GJ):

            @pl.when(jj * BJ <= i * BI)
            def _():
                fo_s_ref[pl.ds(i * BI, BI), :] = jnp.minimum(
                    fo_s_ref[pl.ds(i * BI, BI), :], blockmin(jj)
                )

    @pl.when(t == 2 * G + 1)
    def _():
        sub = jax.lax.broadcasted_iota(jnp.int32, (N, 1), 0)
        isf = (
            (fo_s_ref[...] == sub) & (iso_s_ref[...] == 0)
        ).astype(jnp.bfloat16)
        for b in range(G):
            w = (b + 1) * BI
            fo_b = fo_s_ref[pl.ds(b * BI, BI), :]
            iso_b = iso_s_ref[pl.ds(b * BI, BI), :]
            le = (
                jax.lax.broadcasted_iota(jnp.int32, (BI, w), 1) <= fo_b
            ).astype(jnp.bfloat16)
            r = jnp.dot(
                le, isf[0:w, :], preferred_element_type=jnp.float32
            )
            colors_ref[pl.ds(b * BI, BI), :] = jnp.where(
                iso_b == 1, 0, r.astype(jnp.int32)
            )


def kernel(x, adj_t):
    x32 = x.astype(jnp.int32).reshape(N, 1)
    onehot = (x32 == jnp.arange(NV, dtype=jnp.int32)[None, :]).astype(jnp.float32)

    colors = pl.pallas_call(
        _fused_body,
        grid=(TSTEPS,),
        in_specs=[
            pl.BlockSpec((BI, N), lambda t: (jnp.minimum(t, G - 1), 0)),
            pl.BlockSpec((N, NV), lambda t: (0, 0)),
        ],
        out_specs=pl.BlockSpec((N, 1), lambda t: (0, 0)),
        out_shape=jax.ShapeDtypeStruct((N, 1), jnp.int32),
        scratch_shapes=[
            pltpu.VMEM((N, NV), jnp.float32),
            pltpu.VMEM((1, N), jnp.float32),
            pltpu.VMEM((N, SD), jnp.bfloat16),
            pltpu.VMEM((N, SD), jnp.bfloat16),
            pltpu.VMEM((N, 1), jnp.int32),
            pltpu.VMEM((N, 1), jnp.int32),
        ],
    )(adj_t, onehot)

    return colors.reshape(N).astype(jnp.int64)


# speedup vs baseline: 1.1335x; 1.1335x over previous
import jax
import jax.numpy as jnp
from jax.experimental import pallas as pl
from jax.experimental.pallas import tpu as pltpu

N = 4096
NV = 16
BI = 512
G = N // BI
BJ = 2048
GJ = N // BJ
SD = 40
TSTEPS = G + 1 + G + 1


def _rt_dot(a, b):
    return jax.lax.dot_general(
        a, b, (((1,), (1,)), ((), ())), preferred_element_type=jnp.float32
    )


def _fused_body(adj_ref, oh_ref, colors_ref, counts_ref,
                colsum_ref, t_sig_ref, u_sig_ref, fo_s_ref, iso_s_ref):
    t = pl.program_id(0)

    @pl.when(t < G)
    def _():
        adj = adj_ref[...]
        counts_ref[pl.ds(t * BI, BI), :] = jnp.dot(
            adj, oh_ref[...], preferred_element_type=jnp.float32
        )
        part = jnp.sum(adj, axis=0, keepdims=True)

        @pl.when(t == 0)
        def _():
            colsum_ref[...] = part

        @pl.when(t != 0)
        def _():
            colsum_ref[...] = colsum_ref[...] + part

    @pl.when(t == G)
    def _():
        eye = (
            jax.lax.broadcasted_iota(jnp.int32, (BI, BI), 0)
            == jax.lax.broadcasted_iota(jnp.int32, (BI, BI), 1)
        ).astype(jnp.float32)
        cs_rows = [
            _rt_dot(eye, colsum_ref[:, pl.ds(b * BI, BI)]) for b in range(G)
        ]
        colsum_row = jnp.concatenate(cs_rows, axis=0)

        c = counts_ref[...]
        rowsum = jnp.sum(c, axis=1, keepdims=True)
        iso = (rowsum + colsum_row) == 0
        isof = iso.astype(jnp.float32)
        lane = jax.lax.broadcasted_iota(jnp.int32, (N, NV), 1).astype(
            jnp.float32
        )
        xval = jnp.sum(oh_ref[...] * lane, axis=1, keepdims=True)
        ci = c.astype(jnp.int32)
        hi = (ci >> 6).astype(jnp.float32)
        lo = (ci & 63).astype(jnp.float32)
        nrm = isof + xval * xval + jnp.sum(hi * hi + lo * lo, axis=1,
                                           keepdims=True)
        n2 = jnp.floor(nrm * (1.0 / 4096.0)) * 4096.0
        rem = nrm - n2
        n1 = jnp.floor(rem * (1.0 / 64.0)) * 64.0
        n0 = rem - n1
        ones3 = jnp.ones((N, 3), jnp.float32)
        tmat = jnp.concatenate(
            [2 * isof, 2 * xval, 2 * hi, 2 * lo, -n0, -n1, -n2, ones3], axis=1
        )
        umat = jnp.concatenate(
            [isof, xval, hi, lo, ones3, -n0, -n1, -n2], axis=1
        )
        t_sig_ref[...] = tmat.astype(jnp.bfloat16)
        u_sig_ref[...] = umat.astype(jnp.bfloat16)
        iso_s_ref[...] = iso.astype(jnp.int32)

    @pl.when((t > G) & (t <= 2 * G))
    def _():
        i = t - G - 1
        tb = t_sig_ref[pl.ds(i * BI, BI), :]

        def blockmin(jj):
            g2 = _rt_dot(tb, u_sig_ref[pl.ds(jj * BJ, BJ), :])
            idxj = jax.lax.broadcasted_iota(jnp.int32, (BI, BJ), 1) + jj * BJ
            cand = jnp.where(g2 == 0.0, idxj, jnp.int32(N))
            return jnp.min(cand, axis=1, keepdims=True)

        fo_s_ref[pl.ds(i * BI, BI), :] = blockmin(0)
        for jj in range(1, GJ):

            @pl.when(jj * BJ <= i * BI)
            def _():
                fo_s_ref[pl.ds(i * BI, BI), :] = jnp.minimum(
                    fo_s_ref[pl.ds(i * BI, BI), :], blockmin(jj)
                )

    @pl.when(t == 2 * G + 1)
    def _():
        sub = jax.lax.broadcasted_iota(jnp.int32, (N, 1), 0)
        isf = (
            (fo_s_ref[...] == sub) & (iso_s_ref[...] == 0)
        ).astype(jnp.bfloat16)
        for b in range(G):
            w = (b + 1) * BI
            fo_b = fo_s_ref[pl.ds(b * BI, BI), :]
            iso_b = iso_s_ref[pl.ds(b * BI, BI), :]
            le = (
                jax.lax.broadcasted_iota(jnp.int32, (BI, w), 1) <= fo_b
            ).astype(jnp.bfloat16)
            r = jnp.dot(
                le, isf[0:w, :], preferred_element_type=jnp.float32
            )
            colors_ref[pl.ds(b * BI, BI), :] = jnp.where(
                iso_b == 1, 0, r.astype(jnp.int32)
            )


def kernel(x, adj_t):
    x32 = x.astype(jnp.int32).reshape(N, 1)
    onehot = (x32 == jnp.arange(NV, dtype=jnp.int32)[None, :]).astype(jnp.float32)

    colors = pl.pallas_call(
        _fused_body,
        grid=(TSTEPS,),
        in_specs=[
            pl.BlockSpec((BI, N), lambda t: (jnp.minimum(t, G - 1), 0)),
            pl.BlockSpec((N, NV), lambda t: (0, 0)),
        ],
        out_specs=pl.BlockSpec((N, 1), lambda t: (0, 0)),
        out_shape=jax.ShapeDtypeStruct((N, 1), jnp.int32),
        scratch_shapes=[
            pltpu.VMEM((N, NV), jnp.float32),
            pltpu.VMEM((1, N), jnp.float32),
            pltpu.VMEM((N, SD), jnp.bfloat16),
            pltpu.VMEM((N, SD), jnp.bfloat16),
            pltpu.VMEM((N, 1), jnp.int32),
            pltpu.VMEM((N, 1), jnp.int32),
        ],
    )(adj_t, onehot)

    return colors.reshape(N).astype(jnp.int64)
